# Initial kernel scaffold; baseline (speedup 1.0000x reference)
import jax
import jax.numpy as jnp
from jax.experimental import pallas as pl


def kernel(residue_type, weight):
    raise NotImplementedError("write your pallas kernel here")


# speedup vs baseline: 4.9476x; 4.9476x over previous
import functools

import jax
import jax.numpy as jnp
from jax import lax
from jax.experimental import pallas as pl
from jax.experimental.pallas import tpu as pltpu
from jax.experimental.pallas import tpu_sc as plsc

DIM = 32
NC, NS = 2, 16
NW = NC * NS

K = 16
CW = 128
CHUNK = K * CW


@functools.lru_cache(maxsize=None)
def _make_gather(total):
    n_chunks = total // (CHUNK * NW)
    mesh = plsc.VectorSubcoreMesh(core_axis_name="c", subcore_axis_name="s")

    @functools.partial(
        pl.kernel,
        mesh=mesh,
        out_type=jax.ShapeDtypeStruct((NW * n_chunks, K, CW, DIM), jnp.float32),
        scratch_types=[
            pltpu.VMEM((K, CW), jnp.int32),
            pltpu.VMEM((K, CW, DIM), jnp.float32),
            pltpu.SemaphoreType.DMA,
        ],
        compiler_params=pltpu.CompilerParams(use_tc_tiling_on_sc=False),
    )
    def gather_kernel(table_hbm, idx_hbm, out_hbm, idx_v, rows_v, sem):
        wid = lax.axis_index("s") * NC + lax.axis_index("c")
        base = wid * n_chunks

        def body(g, carry):
            c = base + g
            pltpu.sync_copy(idx_hbm.at[c], idx_v)
            copies = [
                pltpu.async_copy(table_hbm.at[idx_v.at[j]], rows_v.at[j], sem)
                for j in range(K)
            ]
            for cp in copies:
                cp.wait()
            pltpu.sync_copy(rows_v, out_hbm.at[c])
            return carry

        lax.fori_loop(0, n_chunks, body, 0)

    return gather_kernel


def kernel(residue_type, weight):
    b, h = residue_type.shape
    total = b * h
    idx = residue_type.astype(jnp.int32).reshape(-1, K, CW)
    out = _make_gather(total)(weight, idx)
    return out.reshape(b, h, DIM)


# speedup vs baseline: 5.0381x; 1.0183x over previous
import functools

import jax
import jax.numpy as jnp
from jax import lax
from jax.experimental import pallas as pl
from jax.experimental.pallas import tpu as pltpu
from jax.experimental.pallas import tpu_sc as plsc

DIM = 32
NC, NS = 2, 16
NW = NC * NS

K = 10
CW = 128
CHUNK = K * CW


@functools.lru_cache(maxsize=None)
def _make_gather(total):
    n_chunks = total // (CHUNK * NW)
    assert n_chunks % 2 == 0
    mesh = plsc.VectorSubcoreMesh(core_axis_name="c", subcore_axis_name="s")

    @functools.partial(
        pl.kernel,
        mesh=mesh,
        out_type=jax.ShapeDtypeStruct((NW * n_chunks, K, CW, DIM), jnp.float32),
        scratch_types=[
            pltpu.VMEM((K, CW), jnp.int32),
            pltpu.VMEM((K, CW), jnp.int32),
            pltpu.VMEM((K, CW, DIM), jnp.float32),
            pltpu.VMEM((K, CW, DIM), jnp.float32),
            pltpu.SemaphoreType.DMA,
            pltpu.SemaphoreType.DMA,
            pltpu.SemaphoreType.DMA,
            pltpu.SemaphoreType.DMA,
            pltpu.SemaphoreType.DMA,
            pltpu.SemaphoreType.DMA,
        ],
        compiler_params=pltpu.CompilerParams(use_tc_tiling_on_sc=False),
    )
    def gather_kernel(table_hbm, idx_hbm, out_hbm,
                      idx0, idx1, rows0, rows1,
                      si0, si1, sg0, sg1, so0, so1):
        wid = lax.axis_index("s") * NC + lax.axis_index("c")
        base = wid * n_chunks
        idx_v = (idx0, idx1)
        rows_v = (rows0, rows1)
        si = (si0, si1)
        sg = (sg0, sg1)
        so = (so0, so1)

        pltpu.async_copy(idx_hbm.at[base], idx0, si0)
        pltpu.async_copy(idx_hbm.at[base + 1], idx1, si1)

        def step(par, g):
            c = base + g
            @pl.when(g >= 2)
            def _():
                pltpu.make_async_copy(rows_v[par], out_hbm.at[c], so[par]).wait()
            pltpu.make_async_copy(idx_hbm.at[c], idx_v[par], si[par]).wait()
            copies = [
                pltpu.async_copy(
                    table_hbm.at[idx_v[par].at[j]], rows_v[par].at[j], sg[par])
                for j in range(K)
            ]
            for cp in copies:
                cp.wait()
            pltpu.async_copy(rows_v[par], out_hbm.at[c], so[par])
            @pl.when(g + 2 < n_chunks)
            def _():
                pltpu.async_copy(idx_hbm.at[c + 2], idx_v[par], si[par])

        def body(i, carry):
            step(0, 2 * i)
            step(1, 2 * i + 1)
            return carry

        lax.fori_loop(0, n_chunks // 2, body, 0)

        pltpu.make_async_copy(rows0, out_hbm.at[base], so0).wait()
        pltpu.make_async_copy(rows1, out_hbm.at[base], so1).wait()

    return gather_kernel


def kernel(residue_type, weight):
    b, h = residue_type.shape
    total = b * h
    idx = residue_type.astype(jnp.int32).reshape(-1, K, CW)
    out = _make_gather(total)(weight, idx)
    return out.reshape(b, h, DIM)


# speedup vs baseline: 5.0405x; 1.0005x over previous
import functools

import jax
import jax.numpy as jnp
from jax import lax
from jax.experimental import pallas as pl
from jax.experimental.pallas import tpu as pltpu
from jax.experimental.pallas import tpu_sc as plsc

DIM = 32
NC, NS = 2, 16
NW = NC * NS

CHUNK = 1280


@functools.lru_cache(maxsize=None)
def _make_gather(total):
    n_chunks = total // (CHUNK * NW)
    assert n_chunks % 2 == 0
    mesh = plsc.VectorSubcoreMesh(core_axis_name="c", subcore_axis_name="s")

    @functools.partial(
        pl.kernel,
        mesh=mesh,
        out_type=jax.ShapeDtypeStruct((NW * n_chunks, CHUNK, DIM), jnp.float32),
        scratch_types=[
            pltpu.VMEM((CHUNK,), jnp.int32),
            pltpu.VMEM((CHUNK,), jnp.int32),
            pltpu.VMEM((CHUNK, DIM), jnp.float32),
            pltpu.VMEM((CHUNK, DIM), jnp.float32),
            pltpu.SemaphoreType.DMA,
            pltpu.SemaphoreType.DMA,
            pltpu.SemaphoreType.DMA,
            pltpu.SemaphoreType.DMA,
            pltpu.SemaphoreType.DMA,
            pltpu.SemaphoreType.DMA,
        ],
        compiler_params=pltpu.CompilerParams(use_tc_tiling_on_sc=False),
    )
    def gather_kernel(table_hbm, idx_hbm, out_hbm,
                      idx0, idx1, rows0, rows1,
                      si0, si1, sg0, sg1, so0, so1):
        wid = lax.axis_index("s") * NC + lax.axis_index("c")
        base = wid * n_chunks
        idx_v = (idx0, idx1)
        rows_v = (rows0, rows1)
        si = (si0, si1)
        sg = (sg0, sg1)
        so = (so0, so1)

        pltpu.async_copy(idx_hbm.at[base], idx0, si0)
        pltpu.async_copy(idx_hbm.at[base + 1], idx1, si1)

        def step(par, g):
            c = base + g
            @pl.when(g >= 2)
            def _():
                pltpu.make_async_copy(rows_v[par], out_hbm.at[c], so[par]).wait()
            pltpu.make_async_copy(idx_hbm.at[c], idx_v[par], si[par]).wait()
            pltpu.async_copy(
                table_hbm.at[idx_v[par]], rows_v[par], sg[par]).wait()
            pltpu.async_copy(rows_v[par], out_hbm.at[c], so[par])
            @pl.when(g + 2 < n_chunks)
            def _():
                pltpu.async_copy(idx_hbm.at[c + 2], idx_v[par], si[par])

        def body(i, carry):
            step(0, 2 * i)
            step(1, 2 * i + 1)
            return carry

        lax.fori_loop(0, n_chunks // 2, body, 0)

        pltpu.make_async_copy(rows0, out_hbm.at[base], so0).wait()
        pltpu.make_async_copy(rows1, out_hbm.at[base], so1).wait()

    return gather_kernel


def kernel(residue_type, weight):
    b, h = residue_type.shape
    total = b * h
    idx = residue_type.astype(jnp.int32).reshape(-1, CHUNK)
    out = _make_gather(total)(weight, idx)
    return out.reshape(b, h, DIM)
